# Initial kernel scaffold; baseline (speedup 1.0000x reference)
import jax
import jax.numpy as jnp
from jax.experimental import pallas as pl


def kernel(indices, embedding):
    raise NotImplementedError("write your pallas kernel here")


# speedup vs baseline: 6.7910x; 6.7910x over previous
import jax
import jax.numpy as jnp
from jax import lax
from jax.experimental import pallas as pl
from jax.experimental.pallas import tpu as pltpu
from jax.experimental.pallas import tpu_sc as plsc

BATCH = 16384
HIST = 200
NUM_IDX = BATCH * HIST
DIM = 3
PDIM = 8
NC, NS = 2, 16
NW = NC * NS
W = NUM_IDX // NW
CHUNK = 128
BLOCK = 4096
NCHUNK = BLOCK // CHUNK
NBLK = W // BLOCK


def _gather_body(idx_hbm, tab_hbm, out_hbm, idx_v, rows_v, sem_g):
    wid = lax.axis_index("s") * NC + lax.axis_index("c")
    base = wid * W

    def blk(b, carry):
        off = pl.multiple_of(base + b * BLOCK, BLOCK)
        row0 = pl.multiple_of(off // CHUNK, NCHUNK)
        pltpu.sync_copy(idx_hbm.at[pl.ds(row0, NCHUNK)], idx_v)

        def fire(j, c):
            pltpu.async_copy(
                tab_hbm.at[idx_v.at[j]],
                rows_v.at[pl.ds(j * CHUNK, CHUNK)],
                sem_g,
            )
            return c

        lax.fori_loop(0, NCHUNK, fire, 0)

        def drain(j, c):
            pltpu.make_async_copy(
                tab_hbm.at[idx_v.at[j]],
                rows_v.at[pl.ds(j * CHUNK, CHUNK)],
                sem_g,
            ).wait()
            return c

        lax.fori_loop(0, NCHUNK, drain, 0)
        pltpu.sync_copy(rows_v, out_hbm.at[pl.ds(off, BLOCK)])
        return carry

    lax.fori_loop(0, NBLK, blk, 0)


def kernel(indices, embedding):
    idx2d = indices.reshape(NUM_IDX // CHUNK, CHUNK)
    tab = jnp.pad(embedding, ((0, 0), (0, PDIM - DIM)))
    mesh = plsc.VectorSubcoreMesh(core_axis_name="c", subcore_axis_name="s")
    out = pl.kernel(
        _gather_body,
        out_type=jax.ShapeDtypeStruct((NUM_IDX, PDIM), jnp.float32),
        mesh=mesh,
        compiler_params=pltpu.CompilerParams(use_tc_tiling_on_sc=False),
        scratch_types=[
            pltpu.VMEM((NCHUNK, CHUNK), jnp.int32),
            pltpu.VMEM((BLOCK, PDIM), jnp.float32),
            pltpu.SemaphoreType.DMA,
        ],
    )(idx2d, tab)
    return out[:, :DIM].reshape(BATCH, HIST, DIM)


# speedup vs baseline: 48.8877x; 7.1989x over previous
import jax
import jax.numpy as jnp
from jax import lax
from jax.experimental import pallas as pl
from jax.experimental.pallas import tpu as pltpu
from jax.experimental.pallas import tpu_sc as plsc

BATCH = 16384
HIST = 200
NROW = 100000
DIM = 3
NC, NS = 2, 16
NW = NC * NS
BCOL = BATCH // NW
HBLK = 8
NIBLK = HIST // HBLK
VEC = 16


def _lookup_body(idx_hbm, ex_hbm, ey_hbm, ez_hbm, out_hbm,
                 plane_v, idx_v, out_v):
    wid = lax.axis_index("s") * NC + lax.axis_index("c")
    bcol = pl.multiple_of(wid * BCOL, BCOL)

    for d, src in enumerate((ex_hbm, ey_hbm, ez_hbm)):
        pltpu.sync_copy(src, plane_v)

        def iblk(i, c):
            r0 = pl.multiple_of(i * HBLK, HBLK)
            pltpu.sync_copy(idx_hbm.at[pl.ds(r0, HBLK), pl.ds(bcol, BCOL)],
                            idx_v)

            def kstep(k, kc):
                col = pl.multiple_of(k * VEC, VEC)
                for s in range(HBLK):
                    vidx = idx_v[s, pl.ds(col, VEC)]
                    out_v[s, pl.ds(col, VEC)] = plsc.load_gather(
                        plane_v, [vidx])
                return kc

            lax.fori_loop(0, BCOL // VEC, kstep, 0)
            pltpu.sync_copy(out_v,
                            out_hbm.at[d, pl.ds(r0, HBLK), pl.ds(bcol, BCOL)])
            return c

        lax.fori_loop(0, NIBLK, iblk, 0)


def kernel(indices, embedding):
    idx_t = indices.T
    ex = embedding[:, 0]
    ey = embedding[:, 1]
    ez = embedding[:, 2]
    mesh = plsc.VectorSubcoreMesh(core_axis_name="c", subcore_axis_name="s")
    out_t = pl.kernel(
        _lookup_body,
        out_type=jax.ShapeDtypeStruct((DIM, HIST, BATCH), jnp.float32),
        mesh=mesh,
        compiler_params=pltpu.CompilerParams(needs_layout_passes=False),
        scratch_types=[
            pltpu.VMEM((NROW,), jnp.float32),
            pltpu.VMEM((HBLK, BCOL), jnp.int32),
            pltpu.VMEM((HBLK, BCOL), jnp.float32),
        ],
    )(idx_t, ex, ey, ez)
    return out_t.transpose(2, 1, 0)


# speedup vs baseline: 67.6088x; 1.3829x over previous
import jax
import jax.numpy as jnp
from jax import lax
from jax.experimental import pallas as pl
from jax.experimental.pallas import tpu as pltpu
from jax.experimental.pallas import tpu_sc as plsc

BATCH = 16384
HIST = 200
NROW = 100000
DIM = 3
NC, NS = 2, 16
NW = NC * NS
BCOL = BATCH // NW
HBLK = 8
NIBLK = HIST // HBLK
VEC = 16
KUNROLL = 2


def _unit_compute(idx_b, out_b, plane_v):
    def kstep(k, kc):
        col0 = pl.multiple_of(k * VEC * KUNROLL, VEC * KUNROLL)
        for u in range(KUNROLL):
            col = col0 + u * VEC
            for s in range(HBLK):
                vidx = idx_b[s, pl.ds(col, VEC)]
                out_b[s, pl.ds(col, VEC)] = plsc.load_gather(plane_v, [vidx])
        return kc

    lax.fori_loop(0, BCOL // (VEC * KUNROLL), kstep, 0)


def _lookup_body(idx_hbm, ex_hbm, ey_hbm, ez_hbm, out_hbm,
                 plane_v, idx_v, out_v, isem0, isem1, osem0, osem1):
    wid = lax.axis_index("s") * NC + lax.axis_index("c")
    bcol = pl.multiple_of(wid * BCOL, BCOL)
    isems = (isem0, isem1)
    osems = (osem0, osem1)

    def idx_src(i):
        r0 = pl.multiple_of(i * HBLK, HBLK)
        return idx_hbm.at[pl.ds(r0, HBLK), pl.ds(bcol, BCOL)]

    def out_dst(d, i):
        r0 = pl.multiple_of(i * HBLK, HBLK)
        return out_hbm.at[d, pl.ds(r0, HBLK), pl.ds(bcol, BCOL)]

    def step(d, i, b, prefetch_i, wait_store):
        if prefetch_i is not None:
            pltpu.async_copy(idx_src(prefetch_i), idx_v.at[1 - b],
                             isems[1 - b])
        pltpu.make_async_copy(idx_src(i), idx_v.at[b], isems[b]).wait()
        if wait_store:
            pltpu.make_async_copy(out_v.at[b], out_dst(d, i), osems[b]).wait()
        _unit_compute(idx_v.at[b], out_v.at[b], plane_v)
        pltpu.async_copy(out_v.at[b], out_dst(d, i), osems[b])

    for d, src in enumerate((ex_hbm, ey_hbm, ez_hbm)):
        pltpu.sync_copy(src, plane_v)
        pltpu.async_copy(idx_src(0), idx_v.at[0], isems[0])
        step(d, 0, 0, 1, d > 0)
        step(d, 1, 1, 2, d > 0)

        def pair(t, c):
            i0 = pl.multiple_of(2 + 2 * t, 2)
            step(d, i0, 0, i0 + 1, True)
            step(d, i0 + 1, 1, i0 + 2, True)
            return c

        lax.fori_loop(0, (NIBLK - 3) // 2, pair, 0, unroll=False)
        step(d, NIBLK - 1, 0, None, True)

    pltpu.make_async_copy(out_v.at[1], out_dst(DIM - 1, NIBLK - 2),
                          osems[1]).wait()
    pltpu.make_async_copy(out_v.at[0], out_dst(DIM - 1, NIBLK - 1),
                          osems[0]).wait()


def kernel(indices, embedding):
    idx_t = indices.T
    ex = embedding[:, 0]
    ey = embedding[:, 1]
    ez = embedding[:, 2]
    mesh = plsc.VectorSubcoreMesh(core_axis_name="c", subcore_axis_name="s")
    out_t = pl.kernel(
        _lookup_body,
        out_type=jax.ShapeDtypeStruct((DIM, HIST, BATCH), jnp.float32),
        mesh=mesh,
        compiler_params=pltpu.CompilerParams(needs_layout_passes=False),
        scratch_types=[
            pltpu.VMEM((NROW,), jnp.float32),
            pltpu.VMEM((2, HBLK, BCOL), jnp.int32),
            pltpu.VMEM((2, HBLK, BCOL), jnp.float32),
            pltpu.SemaphoreType.DMA,
            pltpu.SemaphoreType.DMA,
            pltpu.SemaphoreType.DMA,
            pltpu.SemaphoreType.DMA,
        ],
    )(idx_t, ex, ey, ez)
    return out_t.transpose(2, 1, 0)


# speedup vs baseline: 115.3411x; 1.7060x over previous
import jax
import jax.numpy as jnp
from jax import lax
from jax.experimental import pallas as pl
from jax.experimental.pallas import tpu as pltpu
from jax.experimental.pallas import tpu_sc as plsc

BATCH = 16384
HIST = 200
NROW = 100000
DIM = 3
NC, NS = 2, 16
NW = NC * NS
BCOL = BATCH // NW
HBLK = 8
NIBLK = HIST // HBLK
VEC = 16
KUNROLL = 2


def _unit_compute(idx_b, out_b, plane_v):
    @plsc.parallel_loop(0, BCOL, step=VEC, unroll=KUNROLL)
    def _(col):
        col = pl.multiple_of(col, VEC)
        for s in range(HBLK):
            vidx = idx_b[s, pl.ds(col, VEC)]
            out_b[s, pl.ds(col, VEC)] = plsc.load_gather(plane_v, [vidx])


def _lookup_body(idx_hbm, ex_hbm, ey_hbm, ez_hbm, out_hbm,
                 plane_v, idx_v, out_v, isem0, isem1, osem0, osem1):
    wid = lax.axis_index("s") * NC + lax.axis_index("c")
    bcol = pl.multiple_of(wid * BCOL, BCOL)
    isems = (isem0, isem1)
    osems = (osem0, osem1)

    def idx_src(i):
        r0 = pl.multiple_of(i * HBLK, HBLK)
        return idx_hbm.at[pl.ds(r0, HBLK), pl.ds(bcol, BCOL)]

    def out_dst(d, i):
        r0 = pl.multiple_of(i * HBLK, HBLK)
        return out_hbm.at[d, pl.ds(r0, HBLK), pl.ds(bcol, BCOL)]

    def step(d, i, b, prefetch_i, wait_store):
        if prefetch_i is not None:
            pltpu.async_copy(idx_src(prefetch_i), idx_v.at[1 - b],
                             isems[1 - b])
        pltpu.make_async_copy(idx_src(i), idx_v.at[b], isems[b]).wait()
        if wait_store:
            pltpu.make_async_copy(out_v.at[b], out_dst(d, i), osems[b]).wait()
        _unit_compute(idx_v.at[b], out_v.at[b], plane_v)
        pltpu.async_copy(out_v.at[b], out_dst(d, i), osems[b])

    for d, src in enumerate((ex_hbm, ey_hbm, ez_hbm)):
        pltpu.sync_copy(src, plane_v)
        pltpu.async_copy(idx_src(0), idx_v.at[0], isems[0])
        step(d, 0, 0, 1, d > 0)
        step(d, 1, 1, 2, d > 0)

        def pair(t, c):
            i0 = pl.multiple_of(2 + 2 * t, 2)
            step(d, i0, 0, i0 + 1, True)
            step(d, i0 + 1, 1, i0 + 2, True)
            return c

        lax.fori_loop(0, (NIBLK - 3) // 2, pair, 0, unroll=False)
        step(d, NIBLK - 1, 0, None, True)

    pltpu.make_async_copy(out_v.at[1], out_dst(DIM - 1, NIBLK - 2),
                          osems[1]).wait()
    pltpu.make_async_copy(out_v.at[0], out_dst(DIM - 1, NIBLK - 1),
                          osems[0]).wait()


def kernel(indices, embedding):
    idx_t = indices.T
    ex = embedding[:, 0]
    ey = embedding[:, 1]
    ez = embedding[:, 2]
    mesh = plsc.VectorSubcoreMesh(core_axis_name="c", subcore_axis_name="s")
    out_t = pl.kernel(
        _lookup_body,
        out_type=jax.ShapeDtypeStruct((DIM, HIST, BATCH), jnp.float32),
        mesh=mesh,
        compiler_params=pltpu.CompilerParams(needs_layout_passes=False),
        scratch_types=[
            pltpu.VMEM((NROW,), jnp.float32),
            pltpu.VMEM((2, HBLK, BCOL), jnp.int32),
            pltpu.VMEM((2, HBLK, BCOL), jnp.float32),
            pltpu.SemaphoreType.DMA,
            pltpu.SemaphoreType.DMA,
            pltpu.SemaphoreType.DMA,
            pltpu.SemaphoreType.DMA,
        ],
    )(idx_t, ex, ey, ez)
    return out_t.transpose(2, 1, 0)
